# baseline (speedup 1.0000x reference)
import jax
import jax.numpy as jnp
from jax import lax
from jax.experimental import pallas as pl
from jax.experimental.pallas import tpu as pltpu
from jax.experimental.pallas import tpu_sc as plsc

_N = 10000
_NP = 10240
_E = 320000
_EP = 327680
_ES = 160000
_ESP = 163840
_W = 128
_BLK = 512
_NSUB = 16
_RPS = _NP // _NSUB
_CW = 16
_F32 = jnp.float32


def _sc_conv(table, src, dst, z2=None):
    feat_split = z2 is not None
    ep = src.shape[0]
    ew = ep // (_NSUB if feat_split else 2 * _NSUB)
    w = 80
    ns = 2 if feat_split else 4
    ring = 2 * ns
    nwin = ew // w

    out_type = jax.ShapeDtypeStruct((2, _NP, 128), _F32)

    scratch = [pltpu.VMEM((w,), jnp.int32)] * (2 * ring)
    scratch += [pltpu.VMEM((w, 128), _F32)] * ns
    if feat_split:
        scratch += [pltpu.VMEM((w, 128), _F32)] * ns
    scratch.append(pltpu.VMEM_SHARED((_NP, 128), _F32))
    nsem = 2 * ns + (ns if feat_split else 0) + ring
    scratch += [pltpu.SemaphoreType.DMA] * nsem

    def body(*refs):
        it = iter(refs)
        table_h = next(it)
        src_h = next(it)
        dst_h = next(it)
        z2_h = next(it) if feat_split else None
        out_s = next(it)
        sidx = tuple(next(it) for _ in range(ring))
        didx = tuple(next(it) for _ in range(ring))
        rows = tuple(next(it) for _ in range(ns))
        z2b = tuple(next(it) for _ in range(ns)) if feat_split else None
        accum = next(it)
        semg = tuple(next(it) for _ in range(ns))
        sems = tuple(next(it) for _ in range(ns))
        semz = tuple(next(it) for _ in range(ns)) if feat_split else None
        semi = tuple(next(it) for _ in range(ring))

        c = lax.axis_index("c")
        s = lax.axis_index("s")
        zero16 = jnp.zeros((16,), _F32)
        off = c * _NP

        def _zr(i, carry):
            for k in range(8):
                rows[0][i, pl.ds(k * 16, 16)] = zero16
            return carry
        lax.fori_loop(0, w, _zr, 0)
        for k in range(_RPS // w):
            pltpu.sync_copy(rows[0], accum.at[pl.ds(s * _RPS + k * w, w)])
        plsc.subcore_barrier()

        wid = s * 2 + c
        base0 = (s if feat_split else wid) * ew

        def load_idx(r, base):
            pltpu.async_copy(src_h.at[pl.ds(base, w)], sidx[r], semi[r])
            pltpu.async_copy(dst_h.at[pl.ds(base, w)], didx[r], semi[r])

        def wait_idx(r):
            pltpu.make_async_copy(src_h.at[pl.ds(0, w)], sidx[r],
                                  semi[r]).wait()
            pltpu.make_async_copy(dst_h.at[pl.ds(0, w)], didx[r],
                                  semi[r]).wait()
            if feat_split:
                for k in range(w // 16):
                    sidx[r][pl.ds(k * 16, 16)] = sidx[r][pl.ds(k * 16, 16)] + off

        def start_fetch(j, r, base):
            pltpu.async_copy(table_h.at[sidx[r]], rows[j], semg[j])
            if feat_split:
                pltpu.async_copy(z2_h.at[pl.ds(c * ep + base, w)], z2b[j],
                                 semz[j])

        def wait_fetch(j, r):
            pltpu.make_async_copy(table_h.at[sidx[r]], rows[j], semg[j]).wait()
            if feat_split:
                pltpu.make_async_copy(z2_h.at[pl.ds(0, w)], z2b[j],
                                      semz[j]).wait()

        def compute(j):
            if feat_split:
                def fr(i, carry):
                    for k in range(8):
                        v = (rows[j][i, pl.ds(k * 16, 16)]
                             + z2b[j][i, pl.ds(k * 16, 16)])
                        rows[j][i, pl.ds(k * 16, 16)] = jnp.maximum(v, 0.0)
                    return carry
                lax.fori_loop(0, w, fr, 0)

        def start_scatter(j, r):
            pltpu.async_copy(rows[j], accum.at[didx[r]], sems[j], add=True)

        def wait_scatter(j):
            pltpu.make_async_copy(rows[j], accum.at[didx[0]], sems[j]).wait()

        for q in range(ns):
            load_idx(q, base0 + q * w)
        wait_idx(0)
        start_fetch(0, 0, base0)

        def group(g, carry):
            n0 = ring * g
            for jj in range(ring):
                n = n0 + jj
                j = jj % ns
                j1 = (jj + 1) % ns
                r = jj % ring
                r1 = (jj + 1) % ring
                rp = (jj + ns) % ring

                @pl.when(n + 1 < nwin)
                def _(j1=j1, r1=r1, n=n):
                    @pl.when(n + 1 >= ns)
                    def _():
                        wait_scatter(j1)
                    wait_idx(r1)
                    start_fetch(j1, r1, base0 + (n + 1) * w)

                @pl.when(n + ns < nwin)
                def _(rp=rp, n=n):
                    load_idx(rp, base0 + (n + ns) * w)
                wait_fetch(j, r)
                compute(j)
                start_scatter(j, r)
            return carry
        lax.fori_loop(0, nwin // ring, group, 0)
        for q in range(ns):
            wait_scatter(q)

        plsc.subcore_barrier()
        pltpu.sync_copy(accum.at[pl.ds(s * _RPS, _RPS)],
                        out_s.at[c, pl.ds(s * _RPS, _RPS)])

    mesh = plsc.VectorSubcoreMesh(core_axis_name="c", subcore_axis_name="s")
    fn = pl.kernel(body, out_type=out_type, mesh=mesh,
                   scratch_types=tuple(scratch))
    args = (table, src, dst) + ((z2,) if feat_split else ())
    return fn(*args)


def _sc_counts(dag_up_dst, dag_dn_dst, slot_dst):
    eps = (dag_up_dst.shape[0], dag_dn_dst.shape[0], slot_dst.shape[0])
    tnp = 3 * _NP
    stripe = tnp // _NSUB
    out_type = jax.ShapeDtypeStruct((2, tnp), _F32)
    scratch = (
        pltpu.VMEM((max(eps) // (2 * _NSUB),), jnp.int32),
        pltpu.VMEM((tnp,), _F32),
        pltpu.VMEM((_NSUB, stripe), _F32),
        pltpu.VMEM((stripe,), _F32),
        pltpu.VMEM_SHARED((_NSUB, tnp), _F32),
    )

    def body(d0, d1, d2, out, didx, ctab, redbuf, sumbuf, shared):
        c = lax.axis_index("c")
        s = lax.axis_index("s")
        wid = s * 2 + c
        zero16 = jnp.zeros((16,), _F32)

        def _zc(i, carry):
            ctab[pl.ds(i * 16, 16)] = zero16
            return carry
        lax.fori_loop(0, tnp // 16, _zc, 0)

        for a, dref, ep in ((0, d0, eps[0]), (1, d1, eps[1]), (2, d2, eps[2])):
            ew = ep // (2 * _NSUB)
            pltpu.sync_copy(dref.at[pl.ds(wid * ew, ew)], didx.at[pl.ds(0, ew)])

            def upd(i, carry, a=a):
                v = didx[pl.ds(i * 16, 16)] + a * _NP
                cnt, msk = plsc.scan_count(v)
                plsc.addupdate_scatter(ctab, [v], cnt.astype(_F32), mask=msk)
                return carry
            lax.fori_loop(0, ew // 16, upd, 0)

        pltpu.sync_copy(ctab, shared.at[s])
        plsc.subcore_barrier()
        for t in range(_NSUB):
            pltpu.sync_copy(shared.at[t, pl.ds(s * stripe, stripe)],
                            redbuf.at[t])

        def red(j, carry):
            acc = redbuf[0, pl.ds(j * 16, 16)]
            for t in range(1, _NSUB):
                acc = acc + redbuf[t, pl.ds(j * 16, 16)]
            sumbuf[pl.ds(j * 16, 16)] = acc
            return carry
        lax.fori_loop(0, stripe // 16, red, 0)
        pltpu.sync_copy(sumbuf, out.at[c, pl.ds(s * stripe, stripe)])

    mesh = plsc.VectorSubcoreMesh(core_axis_name="c", subcore_axis_name="s")
    fn = pl.kernel(body, out_type=out_type, mesh=mesh, scratch_types=scratch,
                   compiler_params=pltpu.CompilerParams(needs_layout_passes=False))
    return fn(dag_up_dst, dag_dn_dst, slot_dst)


def _full(shape):
    return pl.BlockSpec(shape, lambda i: tuple(0 for _ in shape))


def _rows(shape):
    def imap(i):
        return (i,) + tuple(0 for _ in shape[1:])
    return pl.BlockSpec(shape, imap)


def _rows1(shape):
    def imap(i):
        return (0, i) + tuple(0 for _ in shape[2:])
    return pl.BlockSpec(shape, imap)


def _k1(x_ref, wu, bu, wd, bd, yu, yd):
    xb = x_ref[...]
    yu[...] = jnp.maximum(xb @ wu[...] + bu[...], 0.0)
    yd[...] = jnp.maximum(xb @ wd[...] + bd[...], 0.0)


def _k2(x_ref, su_ref, cu_ref, sd_ref, cd_ref,
        wua, wub, wda, wdb, w2, b2, h_ref, y2_ref):
    cnt_u = jnp.maximum(cu_ref[:, 0:1] + cu_ref[:, 1:2], 1.0)
    cnt_d = jnp.maximum(cd_ref[:, 0:1] + cd_ref[:, 1:2], 1.0)
    aggr_u = (su_ref[0] + su_ref[1]) / cnt_u
    aggr_d = (sd_ref[0] + sd_ref[1]) / cnt_d
    xb = x_ref[...]
    xu = jnp.maximum(xb @ wua[...] + aggr_u @ wub[...], 0.0)
    xd = jnp.maximum(xb @ wda[...] + aggr_d @ wdb[...], 0.0)
    h = jnp.concatenate([xu, xd], axis=1)
    h_ref[...] = h
    y2_ref[...] = jnp.maximum(h @ w2[...] + b2[...], 0.0)


def _k3(h_ref, s2_ref, cu_ref, wa, wb, wg, bg, hu_ref, gm_ref):
    i = pl.program_id(0)
    cnt = jnp.maximum(cu_ref[:, 0:1] + cu_ref[:, 1:2], 1.0)
    aggr = (s2_ref[0] + s2_ref[1]) / cnt
    hu = jnp.maximum(h_ref[...] @ wa[...] + aggr @ wb[...], 0.0)
    hu_ref[...] = hu
    g = jnp.maximum(hu @ wg[...] + bg[...], 0.0)
    rowid = i * _BLK + lax.broadcasted_iota(jnp.int32, (_BLK, 1), 0)
    g = jnp.where(rowid < _N, g, 0.0)
    part = jnp.max(g, axis=0, keepdims=True)

    @pl.when(i == 0)
    def _():
        gm_ref[...] = part

    @pl.when(i > 0)
    def _():
        gm_ref[...] = jnp.maximum(gm_ref[...], part)


def _k4a(sx_ref, w1, b1, z1_ref):
    sx = sx_ref[...]
    z1_ref[...] = jnp.stack([sx @ w1[0] + b1[0], sx @ w1[1] + b1[1]], axis=0)


def _k4b(ea_ref, wa, wb, za_ref, zb_ref):
    ea = ea_ref[...]
    za_ref[...] = jnp.stack([ea @ wa[0], ea @ wa[1]], axis=0)
    zb_ref[...] = jnp.stack([ea @ wb[0], ea @ wb[1]], axis=0)


def _k5(sx_ref, sr_ref, cr_ref, wua, wub, w1p, b1p, s1_ref, z1p_ref):
    cnt = jnp.maximum(cr_ref[:, 0:1] + cr_ref[:, 1:2], 1.0)
    aggr = jnp.concatenate([sr_ref[0], sr_ref[1]], axis=1) / cnt
    s1 = jnp.maximum(sx_ref[...] @ wua[...] + aggr @ wub[...], 0.0)
    s1_ref[...] = s1
    z1p_ref[...] = jnp.stack([s1 @ w1p[0] + b1p[0], s1 @ w1p[1] + b1p[1]], 0)


def _k6(s1_ref, sr_ref, cr_ref, wua, wub, so_ref):
    cnt = jnp.maximum(cr_ref[:, 0:1] + cr_ref[:, 1:2], 1.0)
    aggr = jnp.concatenate([sr_ref[0], sr_ref[1]], axis=1) / cnt
    so_ref[...] = jnp.maximum(s1_ref[...] @ wua[...] + aggr @ wub[...], 0.0)


def kernel(x, edge_index, slot_x, slot_edge_index, slot_edge_attr, params):
    p = params
    grid = _NP // _BLK

    xp = jnp.concatenate([x, jnp.zeros((_NP - _N, 128), _F32)], 0)
    sxp = jnp.concatenate([slot_x, jnp.zeros((_NP - _N, 128), _F32)], 0)
    eap = jnp.concatenate([slot_edge_attr, jnp.zeros((_ESP - _ES, 16), _F32)], 0)
    padd = _N + (jnp.arange(_EP - _E, dtype=jnp.int32) % (_NP - _N))
    ei0p = jnp.concatenate([edge_index[0], padd])
    ei1p = jnp.concatenate([edge_index[1], padd])
    spadd = _N + (jnp.arange(_ESP - _ES, dtype=jnp.int32) % (_NP - _N))
    sei0p = jnp.concatenate([slot_edge_index[0], spadd])
    sei1p = jnp.concatenate([slot_edge_index[1], spadd])

    wu1T = p['up1_W'].T
    wd1T = p['down1_W'].T
    bu1 = p['up1_b'][None, :]
    bd1 = p['down1_b'][None, :]
    wu1aT = p['up1_Wu'][:, :128].T
    wu1bT = p['up1_Wu'][:, 128:].T
    wd1aT = p['down1_Wu'][:, :128].T
    wd1bT = p['down1_Wu'][:, 128:].T
    w2T = p['up2_W'].T
    b2 = p['up2_b'][None, :]
    wu2aT = p['up2_Wu'][:, :256].T
    wu2bT = p['up2_Wu'][:, 256:].T
    aggWcT = (p['agg_W'][:, :128] + p['agg_W'][:, 128:]).T
    agg_b = p['agg_b'][None, :]
    r1w1T = p['rc1_W'][:, :128].T
    r1w1s = jnp.stack([r1w1T[:, :128], r1w1T[:, 128:]], 0)
    r1b = p['rc1_b']
    r1bs = jnp.stack([r1b[None, :128], r1b[None, 128:]], 0)
    r1w2T = p['rc1_W'][:, 128:].T
    r1w2s = jnp.stack([r1w2T[:, :128], r1w2T[:, 128:]], 0)
    r2w2T = p['rc2_W'][:, 256:].T
    r2w2s = jnp.stack([r2w2T[:, :128], r2w2T[:, 128:]], 0)
    r1wuaT = p['rc1_Wu'][:, :128].T
    r1wubT = p['rc1_Wu'][:, 128:].T
    r2w1T = p['rc2_W'][:, :256].T
    r2w1s = jnp.stack([r2w1T[:, :128], r2w1T[:, 128:]], 0)
    r2b = p['rc2_b']
    r2bs = jnp.stack([r2b[None, :128], r2b[None, 128:]], 0)
    r2wuaT = p['rc2_Wu'][:, :256].T
    r2wubT = p['rc2_Wu'][:, 256:].T

    cnt_all = _sc_counts(ei1p, ei0p, sei1p).reshape(2, 3, _NP)
    cu = cnt_all[:, 0].T
    cd = cnt_all[:, 1].T
    cs = cnt_all[:, 2].T

    yu, yd = pl.pallas_call(
        _k1,
        grid=(grid,),
        in_specs=[_rows((_BLK, 128)), _full((128, 128)), _full((1, 128)),
                  _full((128, 128)), _full((1, 128))],
        out_specs=[_rows((_BLK, 128)), _rows((_BLK, 128))],
        out_shape=[jax.ShapeDtypeStruct((_NP, 128), _F32)] * 2,
    )(xp, wu1T, bu1, wd1T, bd1)

    su = _sc_conv(yu, ei0p, ei1p)
    sd = _sc_conv(yd, ei1p, ei0p)

    h, y2 = pl.pallas_call(
        _k2,
        grid=(grid,),
        in_specs=[_rows((_BLK, 128)),
                  _rows1((2, _BLK, 128)), _rows((_BLK, 2)),
                  _rows1((2, _BLK, 128)), _rows((_BLK, 2)),
                  _full((128, 128)), _full((128, 128)),
                  _full((128, 128)), _full((128, 128)),
                  _full((256, 128)), _full((1, 128))],
        out_specs=[_rows((_BLK, 256)), _rows((_BLK, 128))],
        out_shape=[jax.ShapeDtypeStruct((_NP, 256), _F32),
                   jax.ShapeDtypeStruct((_NP, 128), _F32)],
    )(xp, su, cu, sd, cd, wu1aT, wu1bT, wd1aT, wd1bT, w2T, b2)

    s2 = _sc_conv(y2, ei0p, ei1p)

    hu, gm = pl.pallas_call(
        _k3,
        grid=(grid,),
        in_specs=[_rows((_BLK, 256)), _rows1((2, _BLK, 128)),
                  _rows((_BLK, 2)),
                  _full((256, 128)), _full((128, 128)),
                  _full((128, 256)), _full((1, 256))],
        out_specs=[_rows((_BLK, 128)), _full((1, 256))],
        out_shape=[jax.ShapeDtypeStruct((_NP, 128), _F32),
                   jax.ShapeDtypeStruct((1, 256), _F32)],
    )(h, s2, cu, wu2aT, wu2bT, aggWcT, agg_b)

    z1 = pl.pallas_call(
        _k4a,
        grid=(grid,),
        in_specs=[_rows((_BLK, 128)), _full((2, 128, 128)), _full((2, 1, 128))],
        out_specs=[_rows1((2, _BLK, 128))],
        out_shape=[jax.ShapeDtypeStruct((2, _NP, 128), _F32)],
    )(sxp, r1w1s, r1bs)[0]

    blke = 1024
    z21, z22 = pl.pallas_call(
        _k4b,
        grid=(_ESP // blke,),
        in_specs=[_rows((blke, 16)), _full((2, 16, 128)), _full((2, 16, 128))],
        out_specs=[_rows1((2, blke, 128)), _rows1((2, blke, 128))],
        out_shape=[jax.ShapeDtypeStruct((2, _ESP, 128), _F32)] * 2,
    )(eap, r1w2s, r2w2s)

    sr1 = _sc_conv(z1.reshape(2 * _NP, 128), sei0p, sei1p,
                   z2=z21.reshape(2 * _ESP, 128))

    s1, z1p = pl.pallas_call(
        _k5,
        grid=(grid,),
        in_specs=[_rows((_BLK, 128)), _rows1((2, _BLK, 128)),
                  _rows((_BLK, 2)),
                  _full((128, 256)), _full((256, 256)),
                  _full((2, 256, 128)), _full((2, 1, 128))],
        out_specs=[_rows((_BLK, 256)), _rows1((2, _BLK, 128))],
        out_shape=[jax.ShapeDtypeStruct((_NP, 256), _F32),
                   jax.ShapeDtypeStruct((2, _NP, 128), _F32)],
    )(sxp, sr1, cs, r1wuaT, r1wubT, r2w1s, r2bs)

    sr2 = _sc_conv(z1p.reshape(2 * _NP, 128), sei0p, sei1p,
                   z2=z22.reshape(2 * _ESP, 128))

    so = pl.pallas_call(
        _k6,
        grid=(grid,),
        in_specs=[_rows((_BLK, 256)), _rows1((2, _BLK, 128)),
                  _rows((_BLK, 2)),
                  _full((256, 256)), _full((256, 256))],
        out_specs=[_rows((_BLK, 256))],
        out_shape=[jax.ShapeDtypeStruct((_NP, 256), _F32)],
    )(s1, sr2, cs, r2wuaT, r2wubT)[0]

    hun = hu[:_N]
    ops_embed = jnp.concatenate([hun, hun], axis=1)
    return ops_embed, gm[0], so[:_N]


# speedup vs baseline: 1.0140x; 1.0140x over previous
import jax
import jax.numpy as jnp
from jax import lax
from jax.experimental import pallas as pl
from jax.experimental.pallas import tpu as pltpu
from jax.experimental.pallas import tpu_sc as plsc

_N = 10000
_NP = 10240
_E = 320000
_EP = 327680
_ES = 160000
_ESP = 163840
_W = 128
_BLK = 512
_NSUB = 16
_RPS = _NP // _NSUB
_CW = 16
_F32 = jnp.float32


def _sc_conv(table, src, dst, z2=None):
    feat_split = z2 is not None
    ep = src.shape[0]
    ew = ep // (_NSUB if feat_split else 2 * _NSUB)
    w = 80 if feat_split else 128
    ns = 2
    ring = 2 * ns
    nwin = ew // w

    out_type = jax.ShapeDtypeStruct((2, _NP, 128), _F32)

    scratch = [pltpu.VMEM((w,), jnp.int32)] * (2 * ring)
    scratch += [pltpu.VMEM((w, 128), _F32)] * ns
    if feat_split:
        scratch += [pltpu.VMEM((w, 128), _F32)] * ns
    scratch.append(pltpu.VMEM_SHARED((_NP, 128), _F32))
    nsem = 2 * ns + (ns if feat_split else 0) + ring
    scratch += [pltpu.SemaphoreType.DMA] * nsem

    def body(*refs):
        it = iter(refs)
        table_h = next(it)
        src_h = next(it)
        dst_h = next(it)
        z2_h = next(it) if feat_split else None
        out_s = next(it)
        sidx = tuple(next(it) for _ in range(ring))
        didx = tuple(next(it) for _ in range(ring))
        rows = tuple(next(it) for _ in range(ns))
        z2b = tuple(next(it) for _ in range(ns)) if feat_split else None
        accum = next(it)
        semg = tuple(next(it) for _ in range(ns))
        sems = tuple(next(it) for _ in range(ns))
        semz = tuple(next(it) for _ in range(ns)) if feat_split else None
        semi = tuple(next(it) for _ in range(ring))

        c = lax.axis_index("c")
        s = lax.axis_index("s")
        zero16 = jnp.zeros((16,), _F32)
        off = c * _NP

        def _zr(i, carry):
            for k in range(8):
                rows[0][i, pl.ds(k * 16, 16)] = zero16
            return carry
        lax.fori_loop(0, w, _zr, 0)
        for k in range(_RPS // w):
            pltpu.sync_copy(rows[0], accum.at[pl.ds(s * _RPS + k * w, w)])
        plsc.subcore_barrier()

        wid = s * 2 + c
        base0 = (s if feat_split else wid) * ew

        def load_idx(r, base):
            pltpu.async_copy(src_h.at[pl.ds(base, w)], sidx[r], semi[r])
            pltpu.async_copy(dst_h.at[pl.ds(base, w)], didx[r], semi[r])

        def wait_idx(r):
            pltpu.make_async_copy(src_h.at[pl.ds(0, w)], sidx[r],
                                  semi[r]).wait()
            pltpu.make_async_copy(dst_h.at[pl.ds(0, w)], didx[r],
                                  semi[r]).wait()
            if feat_split:
                for k in range(w // 16):
                    sidx[r][pl.ds(k * 16, 16)] = sidx[r][pl.ds(k * 16, 16)] + off

        def start_fetch(j, r, base):
            pltpu.async_copy(table_h.at[sidx[r]], rows[j], semg[j])
            if feat_split:
                pltpu.async_copy(z2_h.at[pl.ds(c * ep + base, w)], z2b[j],
                                 semz[j])

        def wait_fetch(j, r):
            pltpu.make_async_copy(table_h.at[sidx[r]], rows[j], semg[j]).wait()
            if feat_split:
                pltpu.make_async_copy(z2_h.at[pl.ds(0, w)], z2b[j],
                                      semz[j]).wait()

        def compute(j):
            if feat_split:
                def fr(i, carry):
                    for k in range(8):
                        v = (rows[j][i, pl.ds(k * 16, 16)]
                             + z2b[j][i, pl.ds(k * 16, 16)])
                        rows[j][i, pl.ds(k * 16, 16)] = jnp.maximum(v, 0.0)
                    return carry
                lax.fori_loop(0, w, fr, 0)

        def start_scatter(j, r):
            pltpu.async_copy(rows[j], accum.at[didx[r]], sems[j], add=True)

        def wait_scatter(j):
            pltpu.make_async_copy(rows[j], accum.at[didx[0]], sems[j]).wait()

        for q in range(ns):
            load_idx(q, base0 + q * w)
        wait_idx(0)
        start_fetch(0, 0, base0)

        def group(g, carry):
            n0 = ring * g
            for jj in range(ring):
                n = n0 + jj
                j = jj % ns
                j1 = (jj + 1) % ns
                r = jj % ring
                r1 = (jj + 1) % ring
                rp = (jj + ns) % ring

                @pl.when(n + 1 < nwin)
                def _(j1=j1, r1=r1, n=n):
                    @pl.when(n + 1 >= ns)
                    def _():
                        wait_scatter(j1)
                    wait_idx(r1)
                    start_fetch(j1, r1, base0 + (n + 1) * w)

                @pl.when(n + ns < nwin)
                def _(rp=rp, n=n):
                    load_idx(rp, base0 + (n + ns) * w)
                wait_fetch(j, r)
                compute(j)
                start_scatter(j, r)
            return carry
        lax.fori_loop(0, nwin // ring, group, 0)
        for q in range(ns):
            wait_scatter(q)

        plsc.subcore_barrier()
        pltpu.sync_copy(accum.at[pl.ds(s * _RPS, _RPS)],
                        out_s.at[c, pl.ds(s * _RPS, _RPS)])

    mesh = plsc.VectorSubcoreMesh(core_axis_name="c", subcore_axis_name="s")
    fn = pl.kernel(body, out_type=out_type, mesh=mesh,
                   scratch_types=tuple(scratch))
    args = (table, src, dst) + ((z2,) if feat_split else ())
    return fn(*args)


def _sc_counts(dag_up_dst, dag_dn_dst, slot_dst):
    eps = (dag_up_dst.shape[0], dag_dn_dst.shape[0], slot_dst.shape[0])
    tnp = 3 * _NP
    stripe = tnp // _NSUB
    out_type = jax.ShapeDtypeStruct((2, tnp), _F32)
    scratch = (
        pltpu.VMEM((max(eps) // (2 * _NSUB),), jnp.int32),
        pltpu.VMEM((tnp,), _F32),
        pltpu.VMEM((_NSUB, stripe), _F32),
        pltpu.VMEM((stripe,), _F32),
        pltpu.VMEM_SHARED((_NSUB, tnp), _F32),
    )

    def body(d0, d1, d2, out, didx, ctab, redbuf, sumbuf, shared):
        c = lax.axis_index("c")
        s = lax.axis_index("s")
        wid = s * 2 + c
        zero16 = jnp.zeros((16,), _F32)

        def _zc(i, carry):
            ctab[pl.ds(i * 16, 16)] = zero16
            return carry
        lax.fori_loop(0, tnp // 16, _zc, 0)

        for a, dref, ep in ((0, d0, eps[0]), (1, d1, eps[1]), (2, d2, eps[2])):
            ew = ep // (2 * _NSUB)
            pltpu.sync_copy(dref.at[pl.ds(wid * ew, ew)], didx.at[pl.ds(0, ew)])

            def upd(i, carry, a=a):
                v = didx[pl.ds(i * 16, 16)] + a * _NP
                cnt, msk = plsc.scan_count(v)
                plsc.addupdate_scatter(ctab, [v], cnt.astype(_F32), mask=msk)
                return carry
            lax.fori_loop(0, ew // 16, upd, 0)

        pltpu.sync_copy(ctab, shared.at[s])
        plsc.subcore_barrier()
        for t in range(_NSUB):
            pltpu.sync_copy(shared.at[t, pl.ds(s * stripe, stripe)],
                            redbuf.at[t])

        def red(j, carry):
            acc = redbuf[0, pl.ds(j * 16, 16)]
            for t in range(1, _NSUB):
                acc = acc + redbuf[t, pl.ds(j * 16, 16)]
            sumbuf[pl.ds(j * 16, 16)] = acc
            return carry
        lax.fori_loop(0, stripe // 16, red, 0)
        pltpu.sync_copy(sumbuf, out.at[c, pl.ds(s * stripe, stripe)])

    mesh = plsc.VectorSubcoreMesh(core_axis_name="c", subcore_axis_name="s")
    fn = pl.kernel(body, out_type=out_type, mesh=mesh, scratch_types=scratch,
                   compiler_params=pltpu.CompilerParams(needs_layout_passes=False))
    return fn(dag_up_dst, dag_dn_dst, slot_dst)


def _full(shape):
    return pl.BlockSpec(shape, lambda i: tuple(0 for _ in shape))


def _rows(shape):
    def imap(i):
        return (i,) + tuple(0 for _ in shape[1:])
    return pl.BlockSpec(shape, imap)


def _rows1(shape):
    def imap(i):
        return (0, i) + tuple(0 for _ in shape[2:])
    return pl.BlockSpec(shape, imap)


def _k1(x_ref, wu, bu, wd, bd, yu, yd):
    xb = x_ref[...]
    yu[...] = jnp.maximum(xb @ wu[...] + bu[...], 0.0)
    yd[...] = jnp.maximum(xb @ wd[...] + bd[...], 0.0)


def _k2(x_ref, su_ref, cu_ref, sd_ref, cd_ref,
        wua, wub, wda, wdb, w2, b2, h_ref, y2_ref):
    cnt_u = jnp.maximum(cu_ref[:, 0:1] + cu_ref[:, 1:2], 1.0)
    cnt_d = jnp.maximum(cd_ref[:, 0:1] + cd_ref[:, 1:2], 1.0)
    aggr_u = (su_ref[0] + su_ref[1]) / cnt_u
    aggr_d = (sd_ref[0] + sd_ref[1]) / cnt_d
    xb = x_ref[...]
    xu = jnp.maximum(xb @ wua[...] + aggr_u @ wub[...], 0.0)
    xd = jnp.maximum(xb @ wda[...] + aggr_d @ wdb[...], 0.0)
    h = jnp.concatenate([xu, xd], axis=1)
    h_ref[...] = h
    y2_ref[...] = jnp.maximum(h @ w2[...] + b2[...], 0.0)


def _k3(h_ref, s2_ref, cu_ref, wa, wb, wg, bg, hu_ref, gm_ref):
    i = pl.program_id(0)
    cnt = jnp.maximum(cu_ref[:, 0:1] + cu_ref[:, 1:2], 1.0)
    aggr = (s2_ref[0] + s2_ref[1]) / cnt
    hu = jnp.maximum(h_ref[...] @ wa[...] + aggr @ wb[...], 0.0)
    hu_ref[...] = hu
    g = jnp.maximum(hu @ wg[...] + bg[...], 0.0)
    rowid = i * _BLK + lax.broadcasted_iota(jnp.int32, (_BLK, 1), 0)
    g = jnp.where(rowid < _N, g, 0.0)
    part = jnp.max(g, axis=0, keepdims=True)

    @pl.when(i == 0)
    def _():
        gm_ref[...] = part

    @pl.when(i > 0)
    def _():
        gm_ref[...] = jnp.maximum(gm_ref[...], part)


def _k4a(sx_ref, w1, b1, z1_ref):
    sx = sx_ref[...]
    z1_ref[...] = jnp.stack([sx @ w1[0] + b1[0], sx @ w1[1] + b1[1]], axis=0)


def _k4b(ea_ref, wa, wb, za_ref, zb_ref):
    ea = ea_ref[...]
    za_ref[...] = jnp.stack([ea @ wa[0], ea @ wa[1]], axis=0)
    zb_ref[...] = jnp.stack([ea @ wb[0], ea @ wb[1]], axis=0)


def _k5(sx_ref, sr_ref, cr_ref, wua, wub, w1p, b1p, s1_ref, z1p_ref):
    cnt = jnp.maximum(cr_ref[:, 0:1] + cr_ref[:, 1:2], 1.0)
    aggr = jnp.concatenate([sr_ref[0], sr_ref[1]], axis=1) / cnt
    s1 = jnp.maximum(sx_ref[...] @ wua[...] + aggr @ wub[...], 0.0)
    s1_ref[...] = s1
    z1p_ref[...] = jnp.stack([s1 @ w1p[0] + b1p[0], s1 @ w1p[1] + b1p[1]], 0)


def _k6(s1_ref, sr_ref, cr_ref, wua, wub, so_ref):
    cnt = jnp.maximum(cr_ref[:, 0:1] + cr_ref[:, 1:2], 1.0)
    aggr = jnp.concatenate([sr_ref[0], sr_ref[1]], axis=1) / cnt
    so_ref[...] = jnp.maximum(s1_ref[...] @ wua[...] + aggr @ wub[...], 0.0)


def kernel(x, edge_index, slot_x, slot_edge_index, slot_edge_attr, params):
    p = params
    grid = _NP // _BLK

    xp = jnp.concatenate([x, jnp.zeros((_NP - _N, 128), _F32)], 0)
    sxp = jnp.concatenate([slot_x, jnp.zeros((_NP - _N, 128), _F32)], 0)
    eap = jnp.concatenate([slot_edge_attr, jnp.zeros((_ESP - _ES, 16), _F32)], 0)
    padd = _N + (jnp.arange(_EP - _E, dtype=jnp.int32) % (_NP - _N))
    ei0p = jnp.concatenate([edge_index[0], padd])
    ei1p = jnp.concatenate([edge_index[1], padd])
    spadd = _N + (jnp.arange(_ESP - _ES, dtype=jnp.int32) % (_NP - _N))
    sei0p = jnp.concatenate([slot_edge_index[0], spadd])
    sei1p = jnp.concatenate([slot_edge_index[1], spadd])

    wu1T = p['up1_W'].T
    wd1T = p['down1_W'].T
    bu1 = p['up1_b'][None, :]
    bd1 = p['down1_b'][None, :]
    wu1aT = p['up1_Wu'][:, :128].T
    wu1bT = p['up1_Wu'][:, 128:].T
    wd1aT = p['down1_Wu'][:, :128].T
    wd1bT = p['down1_Wu'][:, 128:].T
    w2T = p['up2_W'].T
    b2 = p['up2_b'][None, :]
    wu2aT = p['up2_Wu'][:, :256].T
    wu2bT = p['up2_Wu'][:, 256:].T
    aggWcT = (p['agg_W'][:, :128] + p['agg_W'][:, 128:]).T
    agg_b = p['agg_b'][None, :]
    r1w1T = p['rc1_W'][:, :128].T
    r1w1s = jnp.stack([r1w1T[:, :128], r1w1T[:, 128:]], 0)
    r1b = p['rc1_b']
    r1bs = jnp.stack([r1b[None, :128], r1b[None, 128:]], 0)
    r1w2T = p['rc1_W'][:, 128:].T
    r1w2s = jnp.stack([r1w2T[:, :128], r1w2T[:, 128:]], 0)
    r2w2T = p['rc2_W'][:, 256:].T
    r2w2s = jnp.stack([r2w2T[:, :128], r2w2T[:, 128:]], 0)
    r1wuaT = p['rc1_Wu'][:, :128].T
    r1wubT = p['rc1_Wu'][:, 128:].T
    r2w1T = p['rc2_W'][:, :256].T
    r2w1s = jnp.stack([r2w1T[:, :128], r2w1T[:, 128:]], 0)
    r2b = p['rc2_b']
    r2bs = jnp.stack([r2b[None, :128], r2b[None, 128:]], 0)
    r2wuaT = p['rc2_Wu'][:, :256].T
    r2wubT = p['rc2_Wu'][:, 256:].T

    cnt_all = _sc_counts(ei1p, ei0p, sei1p).reshape(2, 3, _NP)
    cu = cnt_all[:, 0].T
    cd = cnt_all[:, 1].T
    cs = cnt_all[:, 2].T

    yu, yd = pl.pallas_call(
        _k1,
        grid=(grid,),
        in_specs=[_rows((_BLK, 128)), _full((128, 128)), _full((1, 128)),
                  _full((128, 128)), _full((1, 128))],
        out_specs=[_rows((_BLK, 128)), _rows((_BLK, 128))],
        out_shape=[jax.ShapeDtypeStruct((_NP, 128), _F32)] * 2,
    )(xp, wu1T, bu1, wd1T, bd1)

    su = _sc_conv(yu, ei0p, ei1p)
    sd = _sc_conv(yd, ei1p, ei0p)

    h, y2 = pl.pallas_call(
        _k2,
        grid=(grid,),
        in_specs=[_rows((_BLK, 128)),
                  _rows1((2, _BLK, 128)), _rows((_BLK, 2)),
                  _rows1((2, _BLK, 128)), _rows((_BLK, 2)),
                  _full((128, 128)), _full((128, 128)),
                  _full((128, 128)), _full((128, 128)),
                  _full((256, 128)), _full((1, 128))],
        out_specs=[_rows((_BLK, 256)), _rows((_BLK, 128))],
        out_shape=[jax.ShapeDtypeStruct((_NP, 256), _F32),
                   jax.ShapeDtypeStruct((_NP, 128), _F32)],
    )(xp, su, cu, sd, cd, wu1aT, wu1bT, wd1aT, wd1bT, w2T, b2)

    s2 = _sc_conv(y2, ei0p, ei1p)

    hu, gm = pl.pallas_call(
        _k3,
        grid=(grid,),
        in_specs=[_rows((_BLK, 256)), _rows1((2, _BLK, 128)),
                  _rows((_BLK, 2)),
                  _full((256, 128)), _full((128, 128)),
                  _full((128, 256)), _full((1, 256))],
        out_specs=[_rows((_BLK, 128)), _full((1, 256))],
        out_shape=[jax.ShapeDtypeStruct((_NP, 128), _F32),
                   jax.ShapeDtypeStruct((1, 256), _F32)],
    )(h, s2, cu, wu2aT, wu2bT, aggWcT, agg_b)

    z1 = pl.pallas_call(
        _k4a,
        grid=(grid,),
        in_specs=[_rows((_BLK, 128)), _full((2, 128, 128)), _full((2, 1, 128))],
        out_specs=[_rows1((2, _BLK, 128))],
        out_shape=[jax.ShapeDtypeStruct((2, _NP, 128), _F32)],
    )(sxp, r1w1s, r1bs)[0]

    blke = 1024
    z21, z22 = pl.pallas_call(
        _k4b,
        grid=(_ESP // blke,),
        in_specs=[_rows((blke, 16)), _full((2, 16, 128)), _full((2, 16, 128))],
        out_specs=[_rows1((2, blke, 128)), _rows1((2, blke, 128))],
        out_shape=[jax.ShapeDtypeStruct((2, _ESP, 128), _F32)] * 2,
    )(eap, r1w2s, r2w2s)

    sr1 = _sc_conv(z1.reshape(2 * _NP, 128), sei0p, sei1p,
                   z2=z21.reshape(2 * _ESP, 128))

    s1, z1p = pl.pallas_call(
        _k5,
        grid=(grid,),
        in_specs=[_rows((_BLK, 128)), _rows1((2, _BLK, 128)),
                  _rows((_BLK, 2)),
                  _full((128, 256)), _full((256, 256)),
                  _full((2, 256, 128)), _full((2, 1, 128))],
        out_specs=[_rows((_BLK, 256)), _rows1((2, _BLK, 128))],
        out_shape=[jax.ShapeDtypeStruct((_NP, 256), _F32),
                   jax.ShapeDtypeStruct((2, _NP, 128), _F32)],
    )(sxp, sr1, cs, r1wuaT, r1wubT, r2w1s, r2bs)

    sr2 = _sc_conv(z1p.reshape(2 * _NP, 128), sei0p, sei1p,
                   z2=z22.reshape(2 * _ESP, 128))

    so = pl.pallas_call(
        _k6,
        grid=(grid,),
        in_specs=[_rows((_BLK, 256)), _rows1((2, _BLK, 128)),
                  _rows((_BLK, 2)),
                  _full((256, 256)), _full((256, 256))],
        out_specs=[_rows((_BLK, 256))],
        out_shape=[jax.ShapeDtypeStruct((_NP, 256), _F32)],
    )(s1, sr2, cs, r2wuaT, r2wubT)[0]

    hun = hu[:_N]
    ops_embed = jnp.concatenate([hun, hun], axis=1)
    return ops_embed, gm[0], so[:_N]
